# Initial kernel scaffold; baseline (speedup 1.0000x reference)
import jax
import jax.numpy as jnp
from jax.experimental import pallas as pl


def kernel(features, batch_nodes, neigh1, neigh2, W_self1, W_neigh1, W_self2, W_neigh2):
    raise NotImplementedError("write your pallas kernel here")


# speedup vs baseline: 6.6604x; 6.6604x over previous
import functools

import jax
import jax.numpy as jnp
from jax import lax
from jax.experimental import pallas as pl
from jax.experimental.pallas import tpu as pltpu
from jax.experimental.pallas import tpu_sc as plsc

D = 128
B = 1024
S1 = 25
S2 = 10
H = 128

NC, NS = 2, 16
NW = NC * NS

G0 = B
G1 = B * S1
SEG = B * S1
OFF_G1 = G0
OFF_S2 = G0 + G1
TOT_ROWS = G0 + G1 + SEG

G0_W = G0 // NW
G1_W = G1 // NW
SEG_W = SEG // NW

CH_SEG = 16
CH_ROWS = CH_SEG * S2
N_CH = SEG_W // CH_SEG
G1_CH = 160
N_G1_CH = G1_W // G1_CH


def _sc_gather(features, idx_all):
  mesh = plsc.VectorSubcoreMesh(core_axis_name="c", subcore_axis_name="s")

  @functools.partial(
      pl.kernel,
      out_type=jax.ShapeDtypeStruct((TOT_ROWS, D), jnp.float32),
      mesh=mesh,
      scratch_types=[
          pltpu.VMEM((SEG_W * S2,), jnp.int32),
          pltpu.VMEM((G1_W,), jnp.int32),
          pltpu.VMEM((G0_W,), jnp.int32),
          pltpu.VMEM((2, CH_ROWS, D), jnp.float32),
          pltpu.VMEM((2, CH_SEG, D), jnp.float32),
          pltpu.VMEM((G1_CH, D), jnp.float32),
          pltpu.VMEM((G0_W, D), jnp.float32),
          pltpu.SemaphoreType.DMA,
          pltpu.SemaphoreType.DMA,
          pltpu.SemaphoreType.DMA,
      ],
  )
  def k(feat_hbm, idx_hbm, out_hbm,
        idx2_v, idx1_v, idx0_v, gbuf, obuf, g1buf, g0buf, sem0, sem1, sem2):
    wid = lax.axis_index("s") * NC + lax.axis_index("c")

    i0 = pl.multiple_of(wid * G0_W, 8)
    i1 = pl.multiple_of(OFF_G1 + wid * G1_W, 8)
    i2 = pl.multiple_of(OFF_S2 + wid * (SEG_W * S2), 8)
    pltpu.sync_copy(idx_hbm.at[pl.ds(i2, SEG_W * S2)], idx2_v)
    pltpu.sync_copy(idx_hbm.at[pl.ds(i1, G1_W)], idx1_v)
    pltpu.sync_copy(idx_hbm.at[pl.ds(i0, G0_W)], idx0_v)

    pltpu.async_copy(feat_hbm.at[idx0_v], g0buf, sem2).wait()
    pltpu.sync_copy(g0buf, out_hbm.at[pl.ds(i0, G0_W)])

    def g1_body(t, carry):
      src = feat_hbm.at[idx1_v.at[pl.ds(pl.multiple_of(t * G1_CH, 8), G1_CH)]]
      pltpu.async_copy(src, g1buf, sem2).wait()
      pltpu.sync_copy(
          g1buf,
          out_hbm.at[pl.ds(pl.multiple_of(OFF_G1 + wid * G1_W + t * G1_CH, 8),
                           G1_CH)])
      return carry
    lax.fori_loop(0, N_G1_CH, g1_body, 0)

    out_base = OFF_S2 + wid * SEG_W
    sems = (sem0, sem1)
    inv = jnp.float32(1.0 / S2)

    def gather_src(g, b):
      off = pl.multiple_of(g * CH_ROWS, 8)
      return (feat_hbm.at[idx2_v.at[pl.ds(off, CH_ROWS)]], gbuf.at[b], sems[b])

    def fire(g, b):
      src, dst, sem = gather_src(g, b)
      pltpu.async_copy(src, dst, sem)

    def wait_g(g, b):
      src, dst, sem = gather_src(g, b)
      pltpu.make_async_copy(src, dst, sem).wait()

    fire(0, 0)
    fire(1, 1)

    def pair_body(p, carry):
      for b in range(2):
        g = p * 2 + b
        wait_g(g, b)

        def seg_body(ci, c2, b=b):
          base = ci * S2
          for j in range(D // 16):
            sl = pl.ds(j * 16, 16)
            acc = gbuf[b, base, sl]
            for r in range(1, S2):
              acc = acc + gbuf[b, base + r, sl]
            obuf[b, ci, sl] = acc * inv
          return c2
        lax.fori_loop(0, CH_SEG, seg_body, 0)

        @pl.when(g + 2 < N_CH)
        def _(g=g, b=b):
          fire(g + 2, b)

        pltpu.sync_copy(
            obuf.at[b],
            out_hbm.at[pl.ds(pl.multiple_of(out_base + g * CH_SEG, 8),
                             CH_SEG)])
      return carry

    lax.fori_loop(0, N_CH // 2, pair_body, 0)

  return k(features, idx_all)


def _tc_dense(sc3, w_s1, w_n1, w_s2, w_n2):
  def body(g1_ref, m2_ref, g0_ref, ws1, wn1, ws2, wn2, out_ref,
           acc_a, acc_b, acc_g):
    s = pl.program_id(0)

    @pl.when(s == 0)
    def _():
      acc_a[...] = jnp.zeros_like(acc_a)
      acc_b[...] = jnp.zeros_like(acc_b)
      acc_g[...] = jnp.zeros_like(acc_g)

    x = g1_ref[0]
    m = m2_ref[0]
    acc_g[...] += x
    acc_a[...] += jnp.maximum(
        jnp.dot(x, ws1[...], preferred_element_type=jnp.float32), 0.0)
    acc_b[...] += jnp.maximum(
        jnp.dot(m, wn1[...], preferred_element_type=jnp.float32), 0.0)

    @pl.when(s == S1 - 1)
    def _():
      inv = jnp.float32(1.0 / S1)
      h1s = jnp.maximum(
          jnp.dot(g0_ref[0], ws1[...], preferred_element_type=jnp.float32),
          0.0)
      h1n = jnp.maximum(
          jnp.dot(acc_g[...] * inv, wn1[...],
                  preferred_element_type=jnp.float32), 0.0)
      w2 = ws2[...]
      self2 = (jnp.dot(h1s, w2[:H], preferred_element_type=jnp.float32)
               + jnp.dot(h1n, w2[H:], preferred_element_type=jnp.float32))
      wn = wn2[...]
      n2 = (jnp.dot(acc_a[...] * inv, wn[:H],
                    preferred_element_type=jnp.float32)
            + jnp.dot(acc_b[...] * inv, wn[H:],
                      preferred_element_type=jnp.float32))
      h2 = jnp.maximum(jnp.concatenate([self2, n2], axis=1), 0.0)
      nrm = jnp.sqrt(jnp.sum(h2 * h2, axis=1, keepdims=True)) + 1e-12
      out_ref[...] = h2 / nrm

  return pl.pallas_call(
      body,
      grid=(S1,),
      in_specs=[
          pl.BlockSpec((1, B, D), lambda s: (1 + s, 0, 0)),
          pl.BlockSpec((1, B, D), lambda s: (1 + S1 + s, 0, 0)),
          pl.BlockSpec((1, B, D), lambda s: (0, 0, 0)),
          pl.BlockSpec((D, H), lambda s: (0, 0)),
          pl.BlockSpec((D, H), lambda s: (0, 0)),
          pl.BlockSpec((2 * H, H), lambda s: (0, 0)),
          pl.BlockSpec((2 * H, H), lambda s: (0, 0)),
      ],
      out_specs=pl.BlockSpec((B, 2 * H), lambda s: (0, 0)),
      out_shape=jax.ShapeDtypeStruct((B, 2 * H), jnp.float32),
      scratch_shapes=[
          pltpu.VMEM((B, H), jnp.float32),
          pltpu.VMEM((B, H), jnp.float32),
          pltpu.VMEM((B, D), jnp.float32),
      ],
  )(sc3, sc3, sc3, w_s1, w_n1, w_s2, w_n2)


def kernel(features, batch_nodes, neigh1, neigh2,
           W_self1, W_neigh1, W_self2, W_neigh2):
  idx_all = jnp.concatenate([
      batch_nodes,
      jnp.transpose(neigh1, (1, 0)).reshape(-1),
      jnp.transpose(neigh2, (1, 0, 2)).reshape(-1),
  ])
  rows = _sc_gather(features, idx_all)
  sc3 = rows.reshape(TOT_ROWS // B, B, D)
  return _tc_dense(sc3, W_self1, W_neigh1, W_self2, W_neigh2)


# speedup vs baseline: 8.1972x; 1.2307x over previous
import functools

import jax
import jax.numpy as jnp
from jax import lax
from jax.experimental import pallas as pl
from jax.experimental.pallas import tpu as pltpu
from jax.experimental.pallas import tpu_sc as plsc

D = 128
B = 1024
S1 = 25
S2 = 10
H = 128

NC, NS = 2, 16
NW = NC * NS

G0 = B
G1 = B * S1
SEG = B * S1
OFF_S2 = G1
OFF_G0 = G1 + SEG
TOT_ROWS = G0 + G1 + SEG

IOFF_G1 = G0
IOFF_S2 = G0 + G1

G0_W = G0 // NW
G1_W = G1 // NW
SEG_W = SEG // NW

CH_SEG = 16
CH_ROWS = CH_SEG * S2
N_CH = SEG_W // CH_SEG
G1_H = G1_W // 2


def _tree_sum(vals):
  while len(vals) > 1:
    nxt = [vals[i] + vals[i + 1] for i in range(0, len(vals) - 1, 2)]
    if len(vals) % 2:
      nxt.append(vals[-1])
    vals = nxt
  return vals[0]


def _sc_gather(features, idx_all):
  mesh = plsc.VectorSubcoreMesh(core_axis_name="c", subcore_axis_name="s")

  @functools.partial(
      pl.kernel,
      out_type=jax.ShapeDtypeStruct((TOT_ROWS, D), jnp.float32),
      mesh=mesh,
      scratch_types=[
          pltpu.VMEM((SEG_W * S2,), jnp.int32),
          pltpu.VMEM((G1_W,), jnp.int32),
          pltpu.VMEM((G0_W,), jnp.int32),
          pltpu.VMEM((2, CH_ROWS, D), jnp.float32),
          pltpu.VMEM((2, CH_SEG, D), jnp.float32),
          pltpu.VMEM((G1_H, D), jnp.float32),
          pltpu.VMEM((G0_W, D), jnp.float32),
          pltpu.SemaphoreType.DMA,
          pltpu.SemaphoreType.DMA,
          pltpu.SemaphoreType.DMA,
          pltpu.SemaphoreType.DMA,
          pltpu.SemaphoreType.DMA,
          pltpu.SemaphoreType.DMA,
      ],
  )
  def k(feat_hbm, idx_hbm, out_hbm,
        idx2_v, idx1_v, idx0_v, gbuf, obuf, g1buf, g0buf,
        gsem0, gsem1, osem0, osem1, g1sem, g0sem):
    wid = lax.axis_index("s") * NC + lax.axis_index("c")

    i0 = pl.multiple_of(wid * G0_W, 8)
    i1 = pl.multiple_of(IOFF_G1 + wid * G1_W, 8)
    i2 = pl.multiple_of(IOFF_S2 + wid * (SEG_W * S2), 8)
    pltpu.sync_copy(idx_hbm.at[pl.ds(i2, SEG_W * S2)], idx2_v)
    pltpu.sync_copy(idx_hbm.at[pl.ds(i1, G1_W)], idx1_v)
    pltpu.sync_copy(idx_hbm.at[pl.ds(i0, G0_W)], idx0_v)

    gsems = (gsem0, gsem1)
    osems = (osem0, osem1)
    mean_base = OFF_S2 + wid * SEG_W
    g1_base = wid * G1_W
    inv = jnp.float32(1.0 / S2)

    def gather_src(g, b):
      off = pl.multiple_of(g * CH_ROWS, 8)
      return (feat_hbm.at[idx2_v.at[pl.ds(off, CH_ROWS)]], gbuf.at[b],
              gsems[b])

    def fire(g, b):
      src, dst, sem = gather_src(g, b)
      pltpu.async_copy(src, dst, sem)

    def wait_gather(g, b):
      src, dst, sem = gather_src(g, b)
      pltpu.make_async_copy(src, dst, sem).wait()

    def mean_out(g, b):
      return (obuf.at[b],
              out_hbm.at[pl.ds(pl.multiple_of(mean_base + g * CH_SEG, 8),
                               CH_SEG)],
              osems[b])

    def g1_half(h):
      src = feat_hbm.at[idx1_v.at[pl.ds(pl.multiple_of(h * G1_H, 8), G1_H)]]
      return src, g1buf, g1sem

    pltpu.async_copy(feat_hbm.at[idx0_v], g0buf, g0sem)
    s, d, sm = g1_half(0)
    pltpu.async_copy(s, d, sm)
    fire(0, 0)
    fire(1, 1)

    def pair_body(p, carry):
      for b in range(2):
        g = p * 2 + b
        wait_gather(g, b)

        @pl.when(p >= 1)
        def _(g=g, b=b):
          src, dst, sem = mean_out(g - 2, b)
          pltpu.make_async_copy(src, dst, sem).wait()

        def seg_body(ci, c2, b=b):
          base = ci * S2
          for j in range(D // 16):
            sl = pl.ds(j * 16, 16)
            acc = _tree_sum([gbuf[b, base + r, sl] for r in range(S2)])
            obuf[b, ci, sl] = acc * inv
          return c2
        lax.fori_loop(0, CH_SEG, seg_body, 0)

        @pl.when(g + 2 < N_CH)
        def _(g=g, b=b):
          fire(g + 2, b)

        src, dst, sem = mean_out(g, b)
        pltpu.async_copy(src, dst, sem)
      return carry

    lax.fori_loop(0, N_CH // 4, pair_body, 0)

    pltpu.make_async_copy(feat_hbm.at[idx0_v], g0buf, g0sem).wait()
    pltpu.sync_copy(g0buf, out_hbm.at[pl.ds(
        pl.multiple_of(OFF_G0 + wid * G0_W, 8), G0_W)])
    s, d, sm = g1_half(0)
    pltpu.make_async_copy(s, d, sm).wait()
    pltpu.sync_copy(g1buf, out_hbm.at[pl.ds(
        pl.multiple_of(g1_base, 8), G1_H)])
    s, d, sm = g1_half(1)
    pltpu.async_copy(s, d, sm)

    lax.fori_loop(N_CH // 4, N_CH // 2, pair_body, 0)

    for b in range(2):
      src, dst, sem = mean_out(N_CH - 2 + b, b)
      pltpu.make_async_copy(src, dst, sem).wait()
    s, d, sm = g1_half(1)
    pltpu.make_async_copy(s, d, sm).wait()
    pltpu.sync_copy(g1buf, out_hbm.at[pl.ds(
        pl.multiple_of(g1_base + G1_H, 8), G1_H)])

  return k(features, idx_all)


TC_BLK = 5


def _tc_dense(sc3, w_s1, w_n1, w_s2, w_n2):
  n_steps = S1 // TC_BLK

  def body(g1_ref, m2_ref, g0_ref, ws1, wn1, ws2, wn2, out_ref,
           acc_a, acc_b, acc_g):
    s = pl.program_id(0)

    @pl.when(s == 0)
    def _():
      acc_a[...] = jnp.zeros_like(acc_a)
      acc_b[...] = jnp.zeros_like(acc_b)
      acc_g[...] = jnp.zeros_like(acc_g)

    a = acc_a[...]
    bb = acc_b[...]
    gg = acc_g[...]
    for r in range(TC_BLK):
      x = g1_ref[r]
      m = m2_ref[r]
      gg = gg + x
      a = a + jnp.maximum(
          jnp.dot(x, ws1[...], preferred_element_type=jnp.float32), 0.0)
      bb = bb + jnp.maximum(
          jnp.dot(m, wn1[...], preferred_element_type=jnp.float32), 0.0)
    acc_a[...] = a
    acc_b[...] = bb
    acc_g[...] = gg

    @pl.when(s == n_steps - 1)
    def _():
      inv = jnp.float32(1.0 / S1)
      h1s = jnp.maximum(
          jnp.dot(g0_ref[0], ws1[...], preferred_element_type=jnp.float32),
          0.0)
      h1n = jnp.maximum(
          jnp.dot(acc_g[...] * inv, wn1[...],
                  preferred_element_type=jnp.float32), 0.0)
      w2 = ws2[...]
      self2 = (jnp.dot(h1s, w2[:H], preferred_element_type=jnp.float32)
               + jnp.dot(h1n, w2[H:], preferred_element_type=jnp.float32))
      wn = wn2[...]
      n2 = (jnp.dot(acc_a[...] * inv, wn[:H],
                    preferred_element_type=jnp.float32)
            + jnp.dot(acc_b[...] * inv, wn[H:],
                      preferred_element_type=jnp.float32))
      h2 = jnp.maximum(jnp.concatenate([self2, n2], axis=1), 0.0)
      nrm = jnp.sqrt(jnp.sum(h2 * h2, axis=1, keepdims=True)) + 1e-12
      out_ref[...] = h2 / nrm

  return pl.pallas_call(
      body,
      grid=(n_steps,),
      in_specs=[
          pl.BlockSpec((TC_BLK, B, D), lambda s: (s, 0, 0)),
          pl.BlockSpec((TC_BLK, B, D), lambda s: (n_steps + s, 0, 0)),
          pl.BlockSpec((1, B, D), lambda s: (2 * S1, 0, 0)),
          pl.BlockSpec((D, H), lambda s: (0, 0)),
          pl.BlockSpec((D, H), lambda s: (0, 0)),
          pl.BlockSpec((2 * H, H), lambda s: (0, 0)),
          pl.BlockSpec((2 * H, H), lambda s: (0, 0)),
      ],
      out_specs=pl.BlockSpec((B, 2 * H), lambda s: (0, 0)),
      out_shape=jax.ShapeDtypeStruct((B, 2 * H), jnp.float32),
      scratch_shapes=[
          pltpu.VMEM((B, H), jnp.float32),
          pltpu.VMEM((B, H), jnp.float32),
          pltpu.VMEM((B, D), jnp.float32),
      ],
  )(sc3, sc3, sc3, w_s1, w_n1, w_s2, w_n2)


def kernel(features, batch_nodes, neigh1, neigh2,
           W_self1, W_neigh1, W_self2, W_neigh2):
  idx_all = jnp.concatenate([
      batch_nodes,
      jnp.transpose(neigh1, (1, 0)).reshape(-1),
      jnp.transpose(neigh2, (1, 0, 2)).reshape(-1),
  ])
  rows = _sc_gather(features, idx_all)
  sc3 = rows.reshape(TOT_ROWS // B, B, D)
  return _tc_dense(sc3, W_self1, W_neigh1, W_self2, W_neigh2)
